# baseline (speedup 1.0000x reference)
import functools

import jax
import jax.numpy as jnp
from jax import lax
from jax.experimental import pallas as pl
from jax.experimental.pallas import tpu as pltpu
from jax.experimental.pallas import tpu_sc as plsc


def kernel(x, token_table, pos_table):
    B, S = x.shape
    V, D = token_table.shape
    assert pos_table.shape == (S, D)

    info = plsc.get_sparse_core_info()
    NC, NS = info.num_cores, info.num_subcores
    NW = NC * NS
    rows_per_w = B // NW
    H = S // 2

    mesh = plsc.VectorSubcoreMesh(core_axis_name="c", subcore_axis_name="s")

    @functools.partial(
        pl.kernel,
        mesh=mesh,
        out_type=jax.ShapeDtypeStruct((B, S, D), jnp.float32),
        scratch_types=[
            pltpu.VMEM((S, D), jnp.float32),
            pltpu.VMEM((2, H), jnp.int32),
            pltpu.VMEM((S, D), jnp.float32),
            pltpu.SemaphoreType.DMA,
        ],
        compiler_params=pltpu.CompilerParams(use_tc_tiling_on_sc=False),
    )
    def tpe(x_hbm, tok_hbm, pos_hbm, out_hbm, pos_v, idx_v, row_v, sem):
        wid = lax.axis_index("s") * NC + lax.axis_index("c")
        base = wid * rows_per_w
        pltpu.sync_copy(pos_hbm, pos_v)

        def body(i, _):
            r = base + i
            pltpu.sync_copy(x_hbm.at[r], idx_v)
            cp0 = pltpu.async_copy(tok_hbm.at[idx_v.at[0]], row_v.at[pl.ds(0, H)], sem)
            cp1 = pltpu.async_copy(tok_hbm.at[idx_v.at[1]], row_v.at[pl.ds(H, H)], sem)
            cp0.wait()
            cp1.wait()

            def add_row(s_, carry):
                for j in range(D // 16):
                    sl = pl.ds(j * 16, 16)
                    row_v[s_, sl] = row_v[s_, sl] + pos_v[s_, sl]
                return carry

            lax.fori_loop(0, S, add_row, 0)
            pltpu.sync_copy(row_v, out_hbm.at[r])
            return _

        lax.fori_loop(0, rows_per_w, body, 0)

    x3 = x.reshape(B, 2, H).astype(jnp.int32)
    return tpe(x3, token_table, pos_table)


# speedup vs baseline: 1.0553x; 1.0553x over previous
import functools

import jax
import jax.numpy as jnp
from jax import lax
from jax.experimental import pallas as pl
from jax.experimental.pallas import tpu as pltpu
from jax.experimental.pallas import tpu_sc as plsc


def kernel(x, token_table, pos_table):
    B, S = x.shape
    V, D = token_table.shape
    assert pos_table.shape == (S, D)

    info = plsc.get_sparse_core_info()
    NC, NS = info.num_cores, info.num_subcores
    NW = NC * NS
    N = B // NW
    NB = 4
    G = N // NB
    H = S // 2

    mesh = plsc.VectorSubcoreMesh(core_axis_name="c", subcore_axis_name="s")

    @functools.partial(
        pl.kernel,
        mesh=mesh,
        out_type=jax.ShapeDtypeStruct((B, S, D), jnp.float32),
        scratch_types=[
            pltpu.VMEM((S, D), jnp.float32),
            pltpu.VMEM((NB, 2, H), jnp.int32),
            pltpu.VMEM((NB, S, D), jnp.float32),
            pltpu.SemaphoreType.DMA,
            pltpu.SemaphoreType.DMA,
            pltpu.SemaphoreType.DMA((NB,)),
        ],
        compiler_params=pltpu.CompilerParams(use_tc_tiling_on_sc=False),
    )
    def tpe(x_hbm, tok_hbm, pos_hbm, out_hbm, pos_v, idx_v, row_v, gsem, isem, ssem):
        wid = lax.axis_index("s") * NC + lax.axis_index("c")
        base = wid * N
        last = base + N - 1
        pltpu.sync_copy(pos_hbm, pos_v)

        def gather_chunks(slot):
            return (
                pltpu.make_async_copy(
                    tok_hbm.at[idx_v.at[slot, 0]], row_v.at[slot, pl.ds(0, H)], gsem),
                pltpu.make_async_copy(
                    tok_hbm.at[idx_v.at[slot, 1]], row_v.at[slot, pl.ds(H, H)], gsem),
            )

        def idx_load(row, slot):
            return pltpu.make_async_copy(
                x_hbm.at[jnp.minimum(row, last)], idx_v.at[slot], isem)

        def store(row, slot):
            return pltpu.make_async_copy(row_v.at[slot], out_hbm.at[row], ssem.at[slot])

        pltpu.sync_copy(x_hbm.at[base], idx_v.at[0])
        for cp in gather_chunks(0):
            cp.start()
        idx_load(base + 1, 1).start()

        def body(g, carry):
            for k in range(NB):
                r = base + g * NB + k
                q = (k + 1) % NB

                for cp in gather_chunks(k):
                    cp.wait()

                if k == NB - 1:
                    store(r - (NB - 1), q).wait()
                else:
                    @pl.when(g > 0)
                    def _():
                        store(r - (NB - 1), q).wait()

                idx_load(r + 1, q).wait()
                if k == NB - 1:
                    @pl.when(g < G - 1)
                    def _():
                        for cp in gather_chunks(q):
                            cp.start()
                else:
                    for cp in gather_chunks(q):
                        cp.start()

                idx_load(r + 2, (k + 2) % NB).start()

                def add_row(s_, c):
                    for j in range(D // 16):
                        sl = pl.ds(j * 16, 16)
                        row_v[k, s_, sl] = row_v[k, s_, sl] + pos_v[s_, sl]
                    return c

                lax.fori_loop(0, S, add_row, 0, unroll=4)

                store(r, k).start()
            return carry

        lax.fori_loop(0, G, body, 0)

        for k in range(1, NB):
            store(base + N - NB + k, k).wait()
        idx_load(last, 1).wait()

    x3 = x.reshape(B, 2, H).astype(jnp.int32)
    return tpe(x3, token_table, pos_table)
